# baseline (speedup 1.0000x reference)
import functools

import numpy as _np

import jax
import jax.numpy as jnp
from jax import lax
from jax.experimental import pallas as pl
from jax.experimental.pallas import tpu as pltpu
from jax.experimental.pallas import tpu_sc as plsc

N = 10000
E = 320000
D = 128
C = 128
A = 32
TAPS = 4

NC = 2
NS = 16
NW = NC * NS
NPAD = 10240
NSLC = NPAD // NS
RPW = 80
EPAD = NW * RPW * 128

_BLK = 2000
_GRID = N // _BLK


def _leaky(x):
    return jnp.where(x >= 0, x, 0.01 * x)


def _bcast_lane(v16, lane):
    idx = jnp.full((16, 1), lane, dtype=jnp.int32)
    return lax.gather(
        v16, idx,
        dimension_numbers=lax.GatherDimensionNumbers(
            offset_dims=(), collapsed_slice_dims=(0,), start_index_map=(0,)),
        slice_sizes=(1,),
        mode=lax.GatherScatterMode.PROMISE_IN_BOUNDS)


_MESH = plsc.VectorSubcoreMesh(core_axis_name="c", subcore_axis_name="s")


def _sc_deg(dstp, wp, zeros1):
    CR = 8

    @functools.partial(
        pl.kernel, mesh=_MESH,
        out_type=jax.ShapeDtypeStruct((NC, NPAD), jnp.float32),
        scratch_types=[
            pltpu.VMEM((CR, 128), jnp.int32),
            pltpu.VMEM((CR, 128), jnp.float32),
            pltpu.VMEM_SHARED((NPAD,), jnp.float32),
        ])
    def k(dst_hbm, w_hbm, z_hbm, out_hbm, didx, wv, acc):
        c = lax.axis_index("c")
        s = lax.axis_index("s")
        wid = s * NC + c
        pltpu.sync_copy(z_hbm.at[pl.ds(s * NSLC, NSLC)],
                        acc.at[pl.ds(s * NSLC, NSLC)])
        plsc.subcore_barrier()

        def chunk(ci, _):
            base = ci * CR
            pltpu.sync_copy(dst_hbm.at[wid, pl.ds(base, CR), :], didx)
            pltpu.sync_copy(w_hbm.at[wid, pl.ds(base, CR), :], wv)
            for j in range(CR):
                pltpu.sync_copy(wv.at[j], acc.at[didx.at[j]], add=True)
            return 0

        lax.fori_loop(0, RPW // CR, chunk, 0)
        plsc.subcore_barrier()
        pltpu.sync_copy(acc.at[pl.ds(s * NSLC, NSLC)],
                        out_hbm.at[c, pl.ds(s * NSLC, NSLC)])

    return k(dstp, wp, zeros1)


def _sc_prop(h, srcp, dstp, normp, zeros2):
    NB = 4
    EW = 64
    SUPER = 40
    NSUB = EPAD // NW // EW
    NQ = SUPER // NB

    @functools.partial(
        pl.kernel, mesh=_MESH,
        out_type=jax.ShapeDtypeStruct((NC, NPAD, C), jnp.float32),
        scratch_types=[
            pltpu.VMEM((SUPER, EW), jnp.int32),
            pltpu.VMEM((SUPER, EW), jnp.int32),
            pltpu.VMEM((SUPER, EW), jnp.float32),
            [pltpu.VMEM((EW, C), jnp.float32) for _ in range(NB)],
            pltpu.VMEM_SHARED((NPAD, C), jnp.float32),
            [pltpu.SemaphoreType.DMA for _ in range(NB)],
            pltpu.SemaphoreType.DMA,
        ])
    def k(h_hbm, src_hbm, dst_hbm, nrm_hbm, z_hbm, out_hbm,
          sidx, didx, nrm, rows, acc, sem_g, sem_s):
        c = lax.axis_index("c")
        s = lax.axis_index("s")
        wid = s * NC + c

        def zrow(e, _):
            for q in range(8):
                rows[0][e, pl.ds(q * 16, 16)] = jnp.zeros((16,), jnp.float32)
            return 0

        lax.fori_loop(0, EW, zrow, 0)

        def zcp(j, _):
            pltpu.sync_copy(rows[0],
                            acc.at[pl.ds(s * NSLC + j * EW, EW), :])
            return 0

        lax.fori_loop(0, NSLC // EW, zcp, 0)
        plsc.subcore_barrier()

        def _drain_scatter():
            pltpu.make_async_copy(z_hbm.at[pl.ds(0, EW), :], rows[0],
                                  sem_s).wait()

        def _scale(buf, row):
            def grp(g, _):
                nv16 = nrm[row, pl.ds(g * 16, 16)]
                for l in range(16):
                    nb = _bcast_lane(nv16, l)
                    e = g * 16 + l
                    for q in range(8):
                        sl = pl.ds(q * 16, 16)
                        buf[e, sl] = buf[e, sl] * nb
                return 0
            lax.fori_loop(0, EW // 16, grp, 0)

        def superchunk(sc, _):
            rbase = sc * SUPER
            pltpu.sync_copy(src_hbm.at[wid, pl.ds(rbase, SUPER), :], sidx)
            pltpu.sync_copy(dst_hbm.at[wid, pl.ds(rbase, SUPER), :], didx)
            pltpu.sync_copy(nrm_hbm.at[wid, pl.ds(rbase, SUPER), :], nrm)
            A = 3
            for u in range(A):
                pltpu.async_copy(h_hbm.at[sidx.at[u]], rows[u], sem_g[u])

            def quad(i, _):
                for u in range(NB):
                    t = i * NB + u
                    pltpu.make_async_copy(h_hbm.at[sidx.at[t]], rows[u],
                                          sem_g[u]).wait()

                    @pl.when(t >= NB - A)
                    def _():
                        _drain_scatter()

                    un = (u + A) % NB

                    @pl.when(t + A <= SUPER - 1)
                    def _():
                        pltpu.async_copy(h_hbm.at[sidx.at[t + A]],
                                         rows[un], sem_g[un])

                    _scale(rows[u], t)
                    pltpu.async_copy(rows[u], acc.at[didx.at[t]], sem_s,
                                     add=True)
                return 0

            lax.fori_loop(0, NQ, quad, 0)
            for _ in range(NB - A):
                _drain_scatter()
            return 0

        lax.fori_loop(0, NSUB // SUPER, superchunk, 0)
        plsc.subcore_barrier()
        pltpu.sync_copy(acc.at[pl.ds(s * NSLC, NSLC), :],
                        out_hbm.at[c, pl.ds(s * NSLC, NSLC), :])

    return k(h, srcp, dstp, normp, zeros2)


def _k_deg_fin(degp_ref, out_ref, rout_ref):
    x = degp_ref[...]
    t = jnp.maximum(x[:NPAD // 128, :] + x[NPAD // 128:, :], 1e-12)
    dv = lax.rsqrt(t)
    out_ref[...] = dv
    rout_ref[...] = jnp.sqrt(t)


def _deg_finish(deg_p):
    nr = NPAD // 128
    return pl.pallas_call(
        _k_deg_fin,
        in_specs=[pl.BlockSpec((2 * nr, 128), lambda: (0, 0))],
        out_specs=[
            pl.BlockSpec((nr, 128), lambda: (0, 0)),
            pl.BlockSpec((nr, 128), lambda: (0, 0)),
        ],
        out_shape=[
            jax.ShapeDtypeStruct((nr, 128), jnp.float32),
            jax.ShapeDtypeStruct((nr, 128), jnp.float32),
        ],
    )(deg_p.reshape(2 * nr, 128))


def _k_read_in(state_ref, w_ref, b_ref, h_ref):
    h_ref[...] = _leaky(
        jnp.dot(state_ref[...], w_ref[...], preferred_element_type=jnp.float32)
        + b_ref[...])


def _read_in(state, W_in, b_in):
    return pl.pallas_call(
        _k_read_in,
        grid=(_GRID,),
        in_specs=[
            pl.BlockSpec((_BLK, D), lambda i: (i, 0)),
            pl.BlockSpec((D, C), lambda i: (0, 0)),
            pl.BlockSpec((1, C), lambda i: (0, 0)),
        ],
        out_specs=pl.BlockSpec((_BLK, C), lambda i: (i, 0)),
        out_shape=jax.ShapeDtypeStruct((N, C), jnp.float32),
    )(state, W_in, b_in.reshape(1, C))


def _k_scale(x_ref, dv_ref, out_ref):
    out_ref[...] = dv_ref[...] * x_ref[...]


def _scale_rows(x, dinv_col):
    return pl.pallas_call(
        _k_scale,
        grid=(_GRID,),
        in_specs=[
            pl.BlockSpec((_BLK, C), lambda i: (i, 0)),
            pl.BlockSpec((_BLK, 1), lambda i: (i, 0)),
        ],
        out_specs=pl.BlockSpec((_BLK, C), lambda i: (i, 0)),
        out_shape=jax.ShapeDtypeStruct((N, C), jnp.float32),
    )(x, dinv_col)


def _p_specs():
    return [
        pl.BlockSpec((1, _BLK, C), lambda i: (0, i, 0)),
        pl.BlockSpec((1, _BLK, C), lambda i: (1, i, 0)),
    ]


def _k_combine(p0_ref, p1_ref, dv_ref, gk_ref):
    dv = dv_ref[...]
    gk_ref[...] = dv * dv * (p0_ref[0] + p1_ref[0])


def _combine(p, dinv_col):
    return pl.pallas_call(
        _k_combine,
        grid=(_GRID,),
        in_specs=[*_p_specs(), pl.BlockSpec((_BLK, 1), lambda i: (i, 0))],
        out_specs=pl.BlockSpec((_BLK, C), lambda i: (i, 0)),
        out_shape=jax.ShapeDtypeStruct((N, C), jnp.float32),
    )(p, p, dinv_col)


def _k_zacc_first(h_ref, g1_ref, rv_ref, w0_ref, w1_ref, z_ref):
    h1 = rv_ref[...] * g1_ref[...]
    z_ref[...] = (
        jnp.dot(h_ref[...], w0_ref[...], preferred_element_type=jnp.float32)
        + jnp.dot(h1, w1_ref[...], preferred_element_type=jnp.float32)
    )


def _zacc_first(h, g1, rdinv_col, W0, W1):
    return pl.pallas_call(
        _k_zacc_first,
        grid=(_GRID,),
        in_specs=[
            pl.BlockSpec((_BLK, C), lambda i: (i, 0)),
            pl.BlockSpec((_BLK, C), lambda i: (i, 0)),
            pl.BlockSpec((_BLK, 1), lambda i: (i, 0)),
            pl.BlockSpec((C, C), lambda i: (0, 0)),
            pl.BlockSpec((C, C), lambda i: (0, 0)),
        ],
        out_specs=pl.BlockSpec((_BLK, C), lambda i: (i, 0)),
        out_shape=jax.ShapeDtypeStruct((N, C), jnp.float32),
    )(h, g1, rdinv_col, W0, W1)


def _k_zacc(z_ref, gk_ref, rv_ref, w_ref, z2_ref):
    hk = rv_ref[...] * gk_ref[...]
    z2_ref[...] = z_ref[...] + jnp.dot(
        hk, w_ref[...], preferred_element_type=jnp.float32)


def _zacc(z, gk, rdinv_col, W):
    return pl.pallas_call(
        _k_zacc,
        grid=(_GRID,),
        in_specs=[
            pl.BlockSpec((_BLK, C), lambda i: (i, 0)),
            pl.BlockSpec((_BLK, C), lambda i: (i, 0)),
            pl.BlockSpec((_BLK, 1), lambda i: (i, 0)),
            pl.BlockSpec((C, C), lambda i: (0, 0)),
        ],
        out_specs=pl.BlockSpec((_BLK, C), lambda i: (i, 0)),
        out_shape=jax.ShapeDtypeStruct((N, C), jnp.float32),
    )(z, gk, rdinv_col, W)


def _k_tap_last(p0_ref, p1_ref, dv_ref, w_ref, z_ref, h_ref, g_ref):
    dv = dv_ref[...]
    hk = dv * (p0_ref[0] + p1_ref[0])
    h = _leaky(z_ref[...] + jnp.dot(
        hk, w_ref[...], preferred_element_type=jnp.float32))
    h_ref[...] = h
    g_ref[...] = dv * h


def _tap_last(p, dinv_col, W, z):
    return pl.pallas_call(
        _k_tap_last,
        grid=(_GRID,),
        in_specs=[
            *_p_specs(),
            pl.BlockSpec((_BLK, 1), lambda i: (i, 0)),
            pl.BlockSpec((C, C), lambda i: (0, 0)),
            pl.BlockSpec((_BLK, C), lambda i: (i, 0)),
        ],
        out_specs=[
            pl.BlockSpec((_BLK, C), lambda i: (i, 0)),
            pl.BlockSpec((_BLK, C), lambda i: (i, 0)),
        ],
        out_shape=[
            jax.ShapeDtypeStruct((N, C), jnp.float32),
            jax.ShapeDtypeStruct((N, C), jnp.float32),
        ],
    )(p, p, dinv_col, W, z)


def _k_read_out(h_ref, w_ref, b_ref, ls_ref, mu_ref, sg_ref):
    mu_ref[...] = jnp.dot(
        h_ref[...], w_ref[...], preferred_element_type=jnp.float32
    ) + b_ref[...]
    sg_ref[...] = jnp.exp(ls_ref[...])


def _read_out(h, W_out, b_out, log_std):
    mu, sg = pl.pallas_call(
        _k_read_out,
        grid=(_GRID,),
        in_specs=[
            pl.BlockSpec((_BLK, C), lambda i: (i, 0)),
            pl.BlockSpec((C, A), lambda i: (0, 0)),
            pl.BlockSpec((1, A), lambda i: (0, 0)),
            pl.BlockSpec((1, A), lambda i: (0, 0)),
        ],
        out_specs=[
            pl.BlockSpec((_BLK, A), lambda i: (i, 0)),
            pl.BlockSpec((1, A), lambda i: (0, 0)),
        ],
        out_shape=[
            jax.ShapeDtypeStruct((N, A), jnp.float32),
            jax.ShapeDtypeStruct((1, A), jnp.float32),
        ],
    )(h, W_out, b_out.reshape(1, A), log_std.reshape(1, A))
    return mu, sg.reshape(A)


def kernel(state, edge_index, edge_attr, W_in, b_in, W_g1, W_g2, W_out, b_out, log_std):
    src = edge_index[0]
    dst = edge_index[1]
    w = edge_attr

    pad = jnp.asarray(_np.arange(EPAD - E, dtype=_np.int32) % N)
    srcp = jnp.concatenate([src, pad]).reshape(NW, RPW, 128)
    dstp = jnp.concatenate([dst, pad]).reshape(NW, RPW, 128)
    wp = jnp.concatenate(
        [w, jnp.zeros((EPAD - E,), jnp.float32)]).reshape(NW, RPW, 128)
    zeros1 = jnp.zeros((NPAD,), jnp.float32)
    zeros2 = jnp.zeros((NPAD, C), jnp.float32)

    deg_p = _sc_deg(dstp, wp, zeros1)
    h = _read_in(state, W_in, b_in)
    dinv2d, rdinv2d = _deg_finish(deg_p)
    dinv_col = dinv2d.reshape(NPAD)[:N].reshape(N, 1)
    rdinv_col = rdinv2d.reshape(NPAD)[:N].reshape(N, 1)

    srcq = srcp.reshape(NW, -1, 64)
    dstq = dstp.reshape(NW, -1, 64)
    wq = wp.reshape(NW, -1, 64)

    g = _scale_rows(h, dinv_col)
    for Wt in (W_g1, W_g2):
        p1 = _sc_prop(g, srcq, dstq, wq, zeros2)
        g1 = _combine(p1, dinv_col)
        p2 = _sc_prop(g1, srcq, dstq, wq, zeros2)
        z = _zacc_first(h, g1, rdinv_col, Wt[0], Wt[1])
        g2 = _combine(p2, dinv_col)
        p3 = _sc_prop(g2, srcq, dstq, wq, zeros2)
        z = _zacc(z, g2, rdinv_col, Wt[2])
        g3 = _combine(p3, dinv_col)
        p4 = _sc_prop(g3, srcq, dstq, wq, zeros2)
        z = _zacc(z, g3, rdinv_col, Wt[3])
        h, g = _tap_last(p4, dinv_col, Wt[TAPS], z)

    return _read_out(h, W_out, b_out, log_std)


# speedup vs baseline: 1.0503x; 1.0503x over previous
import functools

import numpy as _np

import jax
import jax.numpy as jnp
from jax import lax
from jax.experimental import pallas as pl
from jax.experimental.pallas import tpu as pltpu
from jax.experimental.pallas import tpu_sc as plsc

N = 10000
E = 320000
D = 128
C = 128
A = 32
TAPS = 4

NC = 2
NS = 16
NW = NC * NS
NPAD = 10240
NSLC = NPAD // NS
RPW = 80
EPAD = NW * RPW * 128

_BLK = 2000
_GRID = N // _BLK


def _leaky(x):
    return jnp.where(x >= 0, x, 0.01 * x)


def _bcast_lane(v16, lane):
    idx = jnp.full((16, 1), lane, dtype=jnp.int32)
    return lax.gather(
        v16, idx,
        dimension_numbers=lax.GatherDimensionNumbers(
            offset_dims=(), collapsed_slice_dims=(0,), start_index_map=(0,)),
        slice_sizes=(1,),
        mode=lax.GatherScatterMode.PROMISE_IN_BOUNDS)


_MESH = plsc.VectorSubcoreMesh(core_axis_name="c", subcore_axis_name="s")


def _sc_deg(dstp, wp, zeros1):
    CR = 8

    @functools.partial(
        pl.kernel, mesh=_MESH,
        out_type=jax.ShapeDtypeStruct((NC, NPAD), jnp.float32),
        scratch_types=[
            pltpu.VMEM((CR, 128), jnp.int32),
            pltpu.VMEM((CR, 128), jnp.float32),
            pltpu.VMEM_SHARED((NPAD,), jnp.float32),
        ])
    def k(dst_hbm, w_hbm, z_hbm, out_hbm, didx, wv, acc):
        c = lax.axis_index("c")
        s = lax.axis_index("s")
        wid = s * NC + c
        pltpu.sync_copy(z_hbm.at[pl.ds(s * NSLC, NSLC)],
                        acc.at[pl.ds(s * NSLC, NSLC)])
        plsc.subcore_barrier()

        def chunk(ci, _):
            base = ci * CR
            pltpu.sync_copy(dst_hbm.at[wid, pl.ds(base, CR), :], didx)
            pltpu.sync_copy(w_hbm.at[wid, pl.ds(base, CR), :], wv)
            for j in range(CR):
                pltpu.sync_copy(wv.at[j], acc.at[didx.at[j]], add=True)
            return 0

        lax.fori_loop(0, RPW // CR, chunk, 0)
        plsc.subcore_barrier()
        pltpu.sync_copy(acc.at[pl.ds(s * NSLC, NSLC)],
                        out_hbm.at[c, pl.ds(s * NSLC, NSLC)])

    return k(dstp, wp, zeros1)


def _sc_prop(h, srcp, dstp, normp, zeros2):
    NB = 4
    EW = 64
    SUPER = 40
    NSUB = EPAD // NW // EW
    NQ = SUPER // NB

    @functools.partial(
        pl.kernel, mesh=_MESH,
        out_type=jax.ShapeDtypeStruct((NC, NPAD, C), jnp.float32),
        scratch_types=[
            pltpu.VMEM((SUPER, EW), jnp.int32),
            pltpu.VMEM((SUPER, EW), jnp.int32),
            pltpu.VMEM((SUPER, EW), jnp.float32),
            [pltpu.VMEM((EW, C), jnp.float32) for _ in range(NB)],
            pltpu.VMEM_SHARED((NPAD, C), jnp.float32),
            [pltpu.SemaphoreType.DMA for _ in range(NB)],
            pltpu.SemaphoreType.DMA,
        ])
    def k(h_hbm, src_hbm, dst_hbm, nrm_hbm, z_hbm, out_hbm,
          sidx, didx, nrm, rows, acc, sem_g, sem_s):
        c = lax.axis_index("c")
        s = lax.axis_index("s")
        wid = s * NC + c

        def zrow(e, _):
            for q in range(8):
                rows[0][e, pl.ds(q * 16, 16)] = jnp.zeros((16,), jnp.float32)
            return 0

        lax.fori_loop(0, EW, zrow, 0)

        def zcp(j, _):
            pltpu.sync_copy(rows[0],
                            acc.at[pl.ds(s * NSLC + j * EW, EW), :])
            return 0

        lax.fori_loop(0, NSLC // EW, zcp, 0)
        plsc.subcore_barrier()

        def _drain_scatter():
            pltpu.make_async_copy(z_hbm.at[pl.ds(0, EW), :], rows[0],
                                  sem_s).wait()

        def _scale(buf, row):
            def grp(g, _):
                nv16 = nrm[row, pl.ds(g * 16, 16)]
                for l in range(16):
                    nb = _bcast_lane(nv16, l)
                    e = g * 16 + l
                    for q in range(8):
                        sl = pl.ds(q * 16, 16)
                        buf[e, sl] = buf[e, sl] * nb
                return 0
            lax.fori_loop(0, EW // 16, grp, 0)

        def superchunk(sc, _):
            rbase = sc * SUPER
            pltpu.sync_copy(src_hbm.at[wid, pl.ds(rbase, SUPER), :], sidx)
            pltpu.sync_copy(dst_hbm.at[wid, pl.ds(rbase, SUPER), :], didx)
            pltpu.sync_copy(nrm_hbm.at[wid, pl.ds(rbase, SUPER), :], nrm)
            A = 2
            for u in range(A):
                pltpu.async_copy(h_hbm.at[sidx.at[u]], rows[u], sem_g[u])

            def quad(i, _):
                for u in range(NB):
                    t = i * NB + u
                    pltpu.make_async_copy(h_hbm.at[sidx.at[t]], rows[u],
                                          sem_g[u]).wait()

                    @pl.when(t >= NB - A)
                    def _():
                        _drain_scatter()

                    un = (u + A) % NB

                    @pl.when(t + A <= SUPER - 1)
                    def _():
                        pltpu.async_copy(h_hbm.at[sidx.at[t + A]],
                                         rows[un], sem_g[un])

                    _scale(rows[u], t)
                    pltpu.async_copy(rows[u], acc.at[didx.at[t]], sem_s,
                                     add=True)
                return 0

            lax.fori_loop(0, NQ, quad, 0)
            for _ in range(NB - A):
                _drain_scatter()
            return 0

        lax.fori_loop(0, NSUB // SUPER, superchunk, 0)
        plsc.subcore_barrier()
        pltpu.sync_copy(acc.at[pl.ds(s * NSLC, NSLC), :],
                        out_hbm.at[c, pl.ds(s * NSLC, NSLC), :])

    return k(h, srcp, dstp, normp, zeros2)


def _k_deg_fin(degp_ref, out_ref, rout_ref):
    x = degp_ref[...]
    t = jnp.maximum(x[:NPAD // 128, :] + x[NPAD // 128:, :], 1e-12)
    dv = lax.rsqrt(t)
    out_ref[...] = dv
    rout_ref[...] = jnp.sqrt(t)


def _deg_finish(deg_p):
    nr = NPAD // 128
    return pl.pallas_call(
        _k_deg_fin,
        in_specs=[pl.BlockSpec((2 * nr, 128), lambda: (0, 0))],
        out_specs=[
            pl.BlockSpec((nr, 128), lambda: (0, 0)),
            pl.BlockSpec((nr, 128), lambda: (0, 0)),
        ],
        out_shape=[
            jax.ShapeDtypeStruct((nr, 128), jnp.float32),
            jax.ShapeDtypeStruct((nr, 128), jnp.float32),
        ],
    )(deg_p.reshape(2 * nr, 128))


def _k_read_in(state_ref, w_ref, b_ref, h_ref):
    h_ref[...] = _leaky(
        jnp.dot(state_ref[...], w_ref[...], preferred_element_type=jnp.float32)
        + b_ref[...])


def _read_in(state, W_in, b_in):
    return pl.pallas_call(
        _k_read_in,
        grid=(_GRID,),
        in_specs=[
            pl.BlockSpec((_BLK, D), lambda i: (i, 0)),
            pl.BlockSpec((D, C), lambda i: (0, 0)),
            pl.BlockSpec((1, C), lambda i: (0, 0)),
        ],
        out_specs=pl.BlockSpec((_BLK, C), lambda i: (i, 0)),
        out_shape=jax.ShapeDtypeStruct((N, C), jnp.float32),
    )(state, W_in, b_in.reshape(1, C))


def _k_scale(x_ref, dv_ref, out_ref):
    out_ref[...] = dv_ref[...] * x_ref[...]


def _scale_rows(x, dinv_col):
    return pl.pallas_call(
        _k_scale,
        grid=(_GRID,),
        in_specs=[
            pl.BlockSpec((_BLK, C), lambda i: (i, 0)),
            pl.BlockSpec((_BLK, 1), lambda i: (i, 0)),
        ],
        out_specs=pl.BlockSpec((_BLK, C), lambda i: (i, 0)),
        out_shape=jax.ShapeDtypeStruct((N, C), jnp.float32),
    )(x, dinv_col)


def _p_specs():
    return [
        pl.BlockSpec((1, _BLK, C), lambda i: (0, i, 0)),
        pl.BlockSpec((1, _BLK, C), lambda i: (1, i, 0)),
    ]


def _k_combine(p0_ref, p1_ref, dv_ref, gk_ref):
    dv = dv_ref[...]
    gk_ref[...] = dv * dv * (p0_ref[0] + p1_ref[0])


def _combine(p, dinv_col):
    return pl.pallas_call(
        _k_combine,
        grid=(_GRID,),
        in_specs=[*_p_specs(), pl.BlockSpec((_BLK, 1), lambda i: (i, 0))],
        out_specs=pl.BlockSpec((_BLK, C), lambda i: (i, 0)),
        out_shape=jax.ShapeDtypeStruct((N, C), jnp.float32),
    )(p, p, dinv_col)


def _k_zacc_first(h_ref, g1_ref, rv_ref, w0_ref, w1_ref, z_ref):
    h1 = rv_ref[...] * g1_ref[...]
    z_ref[...] = (
        jnp.dot(h_ref[...], w0_ref[...], preferred_element_type=jnp.float32)
        + jnp.dot(h1, w1_ref[...], preferred_element_type=jnp.float32)
    )


def _zacc_first(h, g1, rdinv_col, W0, W1):
    return pl.pallas_call(
        _k_zacc_first,
        grid=(_GRID,),
        in_specs=[
            pl.BlockSpec((_BLK, C), lambda i: (i, 0)),
            pl.BlockSpec((_BLK, C), lambda i: (i, 0)),
            pl.BlockSpec((_BLK, 1), lambda i: (i, 0)),
            pl.BlockSpec((C, C), lambda i: (0, 0)),
            pl.BlockSpec((C, C), lambda i: (0, 0)),
        ],
        out_specs=pl.BlockSpec((_BLK, C), lambda i: (i, 0)),
        out_shape=jax.ShapeDtypeStruct((N, C), jnp.float32),
    )(h, g1, rdinv_col, W0, W1)


def _k_zacc(z_ref, gk_ref, rv_ref, w_ref, z2_ref):
    hk = rv_ref[...] * gk_ref[...]
    z2_ref[...] = z_ref[...] + jnp.dot(
        hk, w_ref[...], preferred_element_type=jnp.float32)


def _zacc(z, gk, rdinv_col, W):
    return pl.pallas_call(
        _k_zacc,
        grid=(_GRID,),
        in_specs=[
            pl.BlockSpec((_BLK, C), lambda i: (i, 0)),
            pl.BlockSpec((_BLK, C), lambda i: (i, 0)),
            pl.BlockSpec((_BLK, 1), lambda i: (i, 0)),
            pl.BlockSpec((C, C), lambda i: (0, 0)),
        ],
        out_specs=pl.BlockSpec((_BLK, C), lambda i: (i, 0)),
        out_shape=jax.ShapeDtypeStruct((N, C), jnp.float32),
    )(z, gk, rdinv_col, W)


def _k_tap_last(p0_ref, p1_ref, dv_ref, w_ref, z_ref, h_ref, g_ref):
    dv = dv_ref[...]
    hk = dv * (p0_ref[0] + p1_ref[0])
    h = _leaky(z_ref[...] + jnp.dot(
        hk, w_ref[...], preferred_element_type=jnp.float32))
    h_ref[...] = h
    g_ref[...] = dv * h


def _tap_last(p, dinv_col, W, z):
    return pl.pallas_call(
        _k_tap_last,
        grid=(_GRID,),
        in_specs=[
            *_p_specs(),
            pl.BlockSpec((_BLK, 1), lambda i: (i, 0)),
            pl.BlockSpec((C, C), lambda i: (0, 0)),
            pl.BlockSpec((_BLK, C), lambda i: (i, 0)),
        ],
        out_specs=[
            pl.BlockSpec((_BLK, C), lambda i: (i, 0)),
            pl.BlockSpec((_BLK, C), lambda i: (i, 0)),
        ],
        out_shape=[
            jax.ShapeDtypeStruct((N, C), jnp.float32),
            jax.ShapeDtypeStruct((N, C), jnp.float32),
        ],
    )(p, p, dinv_col, W, z)


def _k_read_out(h_ref, w_ref, b_ref, ls_ref, mu_ref, sg_ref):
    mu_ref[...] = jnp.dot(
        h_ref[...], w_ref[...], preferred_element_type=jnp.float32
    ) + b_ref[...]
    sg_ref[...] = jnp.exp(ls_ref[...])


def _read_out(h, W_out, b_out, log_std):
    mu, sg = pl.pallas_call(
        _k_read_out,
        grid=(_GRID,),
        in_specs=[
            pl.BlockSpec((_BLK, C), lambda i: (i, 0)),
            pl.BlockSpec((C, A), lambda i: (0, 0)),
            pl.BlockSpec((1, A), lambda i: (0, 0)),
            pl.BlockSpec((1, A), lambda i: (0, 0)),
        ],
        out_specs=[
            pl.BlockSpec((_BLK, A), lambda i: (i, 0)),
            pl.BlockSpec((1, A), lambda i: (0, 0)),
        ],
        out_shape=[
            jax.ShapeDtypeStruct((N, A), jnp.float32),
            jax.ShapeDtypeStruct((1, A), jnp.float32),
        ],
    )(h, W_out, b_out.reshape(1, A), log_std.reshape(1, A))
    return mu, sg.reshape(A)


def kernel(state, edge_index, edge_attr, W_in, b_in, W_g1, W_g2, W_out, b_out, log_std):
    src = edge_index[0]
    dst = edge_index[1]
    w = edge_attr

    pad = jnp.asarray(_np.arange(EPAD - E, dtype=_np.int32) % N)
    srcp = jnp.concatenate([src, pad]).reshape(NW, RPW, 128)
    dstp = jnp.concatenate([dst, pad]).reshape(NW, RPW, 128)
    wp = jnp.concatenate(
        [w, jnp.zeros((EPAD - E,), jnp.float32)]).reshape(NW, RPW, 128)
    zeros1 = jnp.zeros((NPAD,), jnp.float32)
    zeros2 = jnp.zeros((NPAD, C), jnp.float32)

    deg_p = _sc_deg(dstp, wp, zeros1)
    h = _read_in(state, W_in, b_in)
    dinv2d, rdinv2d = _deg_finish(deg_p)
    dinv_col = dinv2d.reshape(NPAD)[:N].reshape(N, 1)
    rdinv_col = rdinv2d.reshape(NPAD)[:N].reshape(N, 1)

    srcq = srcp.reshape(NW, -1, 64)
    dstq = dstp.reshape(NW, -1, 64)
    wq = wp.reshape(NW, -1, 64)

    g = _scale_rows(h, dinv_col)
    for Wt in (W_g1, W_g2):
        p1 = _sc_prop(g, srcq, dstq, wq, zeros2)
        g1 = _combine(p1, dinv_col)
        p2 = _sc_prop(g1, srcq, dstq, wq, zeros2)
        z = _zacc_first(h, g1, rdinv_col, Wt[0], Wt[1])
        g2 = _combine(p2, dinv_col)
        p3 = _sc_prop(g2, srcq, dstq, wq, zeros2)
        z = _zacc(z, g2, rdinv_col, Wt[2])
        g3 = _combine(p3, dinv_col)
        p4 = _sc_prop(g3, srcq, dstq, wq, zeros2)
        z = _zacc(z, g3, rdinv_col, Wt[3])
        h, g = _tap_last(p4, dinv_col, Wt[TAPS], z)

    return _read_out(h, W_out, b_out, log_std)


# speedup vs baseline: 1.0868x; 1.0347x over previous
import functools

import numpy as _np

import jax
import jax.numpy as jnp
from jax import lax
from jax.experimental import pallas as pl
from jax.experimental.pallas import tpu as pltpu
from jax.experimental.pallas import tpu_sc as plsc

N = 10000
E = 320000
D = 128
C = 128
A = 32
TAPS = 4

NC = 2
NS = 16
NW = NC * NS
NPAD = 10240
NSLC = NPAD // NS
RPW = 80
EPAD = NW * RPW * 128

_BLK = 2000
_GRID = N // _BLK


def _leaky(x):
    return jnp.where(x >= 0, x, 0.01 * x)


def _bcast_lane(v16, lane):
    idx = jnp.full((16, 1), lane, dtype=jnp.int32)
    return lax.gather(
        v16, idx,
        dimension_numbers=lax.GatherDimensionNumbers(
            offset_dims=(), collapsed_slice_dims=(0,), start_index_map=(0,)),
        slice_sizes=(1,),
        mode=lax.GatherScatterMode.PROMISE_IN_BOUNDS)


_MESH = plsc.VectorSubcoreMesh(core_axis_name="c", subcore_axis_name="s")


def _sc_deg(dstp, wp, zeros1):
    CR = 8

    @functools.partial(
        pl.kernel, mesh=_MESH,
        out_type=jax.ShapeDtypeStruct((NC, NPAD), jnp.float32),
        scratch_types=[
            pltpu.VMEM((CR, 128), jnp.int32),
            pltpu.VMEM((CR, 128), jnp.float32),
            pltpu.VMEM_SHARED((NPAD,), jnp.float32),
        ])
    def k(dst_hbm, w_hbm, z_hbm, out_hbm, didx, wv, acc):
        c = lax.axis_index("c")
        s = lax.axis_index("s")
        wid = s * NC + c
        pltpu.sync_copy(z_hbm.at[pl.ds(s * NSLC, NSLC)],
                        acc.at[pl.ds(s * NSLC, NSLC)])
        plsc.subcore_barrier()

        def chunk(ci, _):
            base = ci * CR
            pltpu.sync_copy(dst_hbm.at[wid, pl.ds(base, CR), :], didx)
            pltpu.sync_copy(w_hbm.at[wid, pl.ds(base, CR), :], wv)
            for j in range(CR):
                pltpu.sync_copy(wv.at[j], acc.at[didx.at[j]], add=True)
            return 0

        lax.fori_loop(0, RPW // CR, chunk, 0)
        plsc.subcore_barrier()
        pltpu.sync_copy(acc.at[pl.ds(s * NSLC, NSLC)],
                        out_hbm.at[c, pl.ds(s * NSLC, NSLC)])

    return k(dstp, wp, zeros1)


def _sc_prop(h, srcp, dstp, normp, zeros2):
    NB = 4
    EW = 64
    SUPER = 40
    NSUB = EPAD // NW // EW
    NQ = SUPER // NB

    @functools.partial(
        pl.kernel, mesh=_MESH,
        out_type=jax.ShapeDtypeStruct((NC, NPAD, C), jnp.float32),
        scratch_types=[
            pltpu.VMEM((SUPER, EW), jnp.int32),
            pltpu.VMEM((SUPER, EW), jnp.int32),
            pltpu.VMEM((SUPER, EW), jnp.float32),
            [pltpu.VMEM((EW, C), jnp.float32) for _ in range(NB)],
            pltpu.VMEM_SHARED((NPAD, C), jnp.float32),
            [pltpu.SemaphoreType.DMA for _ in range(NB)],
            pltpu.SemaphoreType.DMA,
        ])
    def k(h_hbm, src_hbm, dst_hbm, nrm_hbm, z_hbm, out_hbm,
          sidx, didx, nrm, rows, acc, sem_g, sem_s):
        c = lax.axis_index("c")
        s = lax.axis_index("s")
        wid = s * NC + c

        def zrow(e, _):
            for q in range(8):
                rows[0][e, pl.ds(q * 16, 16)] = jnp.zeros((16,), jnp.float32)
            return 0

        lax.fori_loop(0, EW, zrow, 0)

        def zcp(j, _):
            pltpu.sync_copy(rows[0],
                            acc.at[pl.ds(s * NSLC + j * EW, EW), :])
            return 0

        lax.fori_loop(0, NSLC // EW, zcp, 0)
        plsc.subcore_barrier()

        def _drain_scatter():
            pltpu.make_async_copy(z_hbm.at[pl.ds(0, EW), :], rows[0],
                                  sem_s).wait()

        def _scale(buf, row):
            def grp(g, _):
                nv16 = nrm[row, pl.ds(g * 16, 16)]
                for l in range(16):
                    nb = _bcast_lane(nv16, l)
                    e = g * 16 + l
                    for q in range(8):
                        sl = pl.ds(q * 16, 16)
                        buf[e, sl] = buf[e, sl] * nb
                return 0
            lax.fori_loop(0, EW // 16, grp, 0)

        def superchunk(sc, _):
            rbase = sc * SUPER
            st = [pltpu.async_copy(src_hbm.at[wid, pl.ds(rbase, SUPER), :],
                                   sidx, sem_g[0]),
                  pltpu.async_copy(dst_hbm.at[wid, pl.ds(rbase, SUPER), :],
                                   didx, sem_g[1]),
                  pltpu.async_copy(nrm_hbm.at[wid, pl.ds(rbase, SUPER), :],
                                   nrm, sem_g[2])]
            for h_ in st:
                h_.wait()
            A = 2
            for u in range(A):
                pltpu.async_copy(h_hbm.at[sidx.at[u]], rows[u], sem_g[u])

            def quad(i, _):
                for u in range(NB):
                    t = i * NB + u
                    pltpu.make_async_copy(h_hbm.at[sidx.at[t]], rows[u],
                                          sem_g[u]).wait()

                    @pl.when(t >= NB - A)
                    def _():
                        _drain_scatter()

                    un = (u + A) % NB

                    @pl.when(t + A <= SUPER - 1)
                    def _():
                        pltpu.async_copy(h_hbm.at[sidx.at[t + A]],
                                         rows[un], sem_g[un])

                    _scale(rows[u], t)
                    pltpu.async_copy(rows[u], acc.at[didx.at[t]], sem_s,
                                     add=True)
                return 0

            lax.fori_loop(0, NQ, quad, 0)
            for _ in range(NB - A):
                _drain_scatter()
            return 0

        lax.fori_loop(0, NSUB // SUPER, superchunk, 0)
        plsc.subcore_barrier()
        pltpu.sync_copy(acc.at[pl.ds(s * NSLC, NSLC), :],
                        out_hbm.at[c, pl.ds(s * NSLC, NSLC), :])

    return k(h, srcp, dstp, normp, zeros2)


def _k_deg_fin(degp_ref, out_ref, rout_ref):
    x = degp_ref[...]
    t = jnp.maximum(x[:NPAD // 128, :] + x[NPAD // 128:, :], 1e-12)
    dv = lax.rsqrt(t)
    out_ref[...] = dv
    rout_ref[...] = jnp.sqrt(t)


def _deg_finish(deg_p):
    nr = NPAD // 128
    return pl.pallas_call(
        _k_deg_fin,
        in_specs=[pl.BlockSpec((2 * nr, 128), lambda: (0, 0))],
        out_specs=[
            pl.BlockSpec((nr, 128), lambda: (0, 0)),
            pl.BlockSpec((nr, 128), lambda: (0, 0)),
        ],
        out_shape=[
            jax.ShapeDtypeStruct((nr, 128), jnp.float32),
            jax.ShapeDtypeStruct((nr, 128), jnp.float32),
        ],
    )(deg_p.reshape(2 * nr, 128))


def _k_read_in(state_ref, w_ref, b_ref, h_ref):
    h_ref[...] = _leaky(
        jnp.dot(state_ref[...], w_ref[...], preferred_element_type=jnp.float32)
        + b_ref[...])


def _read_in(state, W_in, b_in):
    return pl.pallas_call(
        _k_read_in,
        grid=(_GRID,),
        in_specs=[
            pl.BlockSpec((_BLK, D), lambda i: (i, 0)),
            pl.BlockSpec((D, C), lambda i: (0, 0)),
            pl.BlockSpec((1, C), lambda i: (0, 0)),
        ],
        out_specs=pl.BlockSpec((_BLK, C), lambda i: (i, 0)),
        out_shape=jax.ShapeDtypeStruct((N, C), jnp.float32),
    )(state, W_in, b_in.reshape(1, C))


def _k_scale(x_ref, dv_ref, out_ref):
    out_ref[...] = dv_ref[...] * x_ref[...]


def _scale_rows(x, dinv_col):
    return pl.pallas_call(
        _k_scale,
        grid=(_GRID,),
        in_specs=[
            pl.BlockSpec((_BLK, C), lambda i: (i, 0)),
            pl.BlockSpec((_BLK, 1), lambda i: (i, 0)),
        ],
        out_specs=pl.BlockSpec((_BLK, C), lambda i: (i, 0)),
        out_shape=jax.ShapeDtypeStruct((N, C), jnp.float32),
    )(x, dinv_col)


def _p_specs():
    return [
        pl.BlockSpec((1, _BLK, C), lambda i: (0, i, 0)),
        pl.BlockSpec((1, _BLK, C), lambda i: (1, i, 0)),
    ]


def _k_combine(p0_ref, p1_ref, dv_ref, gk_ref):
    dv = dv_ref[...]
    gk_ref[...] = dv * dv * (p0_ref[0] + p1_ref[0])


def _combine(p, dinv_col):
    return pl.pallas_call(
        _k_combine,
        grid=(_GRID,),
        in_specs=[*_p_specs(), pl.BlockSpec((_BLK, 1), lambda i: (i, 0))],
        out_specs=pl.BlockSpec((_BLK, C), lambda i: (i, 0)),
        out_shape=jax.ShapeDtypeStruct((N, C), jnp.float32),
    )(p, p, dinv_col)


def _k_zacc_first(h_ref, g1_ref, rv_ref, w0_ref, w1_ref, z_ref):
    h1 = rv_ref[...] * g1_ref[...]
    z_ref[...] = (
        jnp.dot(h_ref[...], w0_ref[...], preferred_element_type=jnp.float32)
        + jnp.dot(h1, w1_ref[...], preferred_element_type=jnp.float32)
    )


def _zacc_first(h, g1, rdinv_col, W0, W1):
    return pl.pallas_call(
        _k_zacc_first,
        grid=(_GRID,),
        in_specs=[
            pl.BlockSpec((_BLK, C), lambda i: (i, 0)),
            pl.BlockSpec((_BLK, C), lambda i: (i, 0)),
            pl.BlockSpec((_BLK, 1), lambda i: (i, 0)),
            pl.BlockSpec((C, C), lambda i: (0, 0)),
            pl.BlockSpec((C, C), lambda i: (0, 0)),
        ],
        out_specs=pl.BlockSpec((_BLK, C), lambda i: (i, 0)),
        out_shape=jax.ShapeDtypeStruct((N, C), jnp.float32),
    )(h, g1, rdinv_col, W0, W1)


def _k_zacc(z_ref, gk_ref, rv_ref, w_ref, z2_ref):
    hk = rv_ref[...] * gk_ref[...]
    z2_ref[...] = z_ref[...] + jnp.dot(
        hk, w_ref[...], preferred_element_type=jnp.float32)


def _zacc(z, gk, rdinv_col, W):
    return pl.pallas_call(
        _k_zacc,
        grid=(_GRID,),
        in_specs=[
            pl.BlockSpec((_BLK, C), lambda i: (i, 0)),
            pl.BlockSpec((_BLK, C), lambda i: (i, 0)),
            pl.BlockSpec((_BLK, 1), lambda i: (i, 0)),
            pl.BlockSpec((C, C), lambda i: (0, 0)),
        ],
        out_specs=pl.BlockSpec((_BLK, C), lambda i: (i, 0)),
        out_shape=jax.ShapeDtypeStruct((N, C), jnp.float32),
    )(z, gk, rdinv_col, W)


def _k_tap_last(p0_ref, p1_ref, dv_ref, w_ref, z_ref, h_ref, g_ref):
    dv = dv_ref[...]
    hk = dv * (p0_ref[0] + p1_ref[0])
    h = _leaky(z_ref[...] + jnp.dot(
        hk, w_ref[...], preferred_element_type=jnp.float32))
    h_ref[...] = h
    g_ref[...] = dv * h


def _tap_last(p, dinv_col, W, z):
    return pl.pallas_call(
        _k_tap_last,
        grid=(_GRID,),
        in_specs=[
            *_p_specs(),
            pl.BlockSpec((_BLK, 1), lambda i: (i, 0)),
            pl.BlockSpec((C, C), lambda i: (0, 0)),
            pl.BlockSpec((_BLK, C), lambda i: (i, 0)),
        ],
        out_specs=[
            pl.BlockSpec((_BLK, C), lambda i: (i, 0)),
            pl.BlockSpec((_BLK, C), lambda i: (i, 0)),
        ],
        out_shape=[
            jax.ShapeDtypeStruct((N, C), jnp.float32),
            jax.ShapeDtypeStruct((N, C), jnp.float32),
        ],
    )(p, p, dinv_col, W, z)


def _k_read_out(h_ref, w_ref, b_ref, ls_ref, mu_ref, sg_ref):
    mu_ref[...] = jnp.dot(
        h_ref[...], w_ref[...], preferred_element_type=jnp.float32
    ) + b_ref[...]
    sg_ref[...] = jnp.exp(ls_ref[...])


def _read_out(h, W_out, b_out, log_std):
    mu, sg = pl.pallas_call(
        _k_read_out,
        grid=(_GRID,),
        in_specs=[
            pl.BlockSpec((_BLK, C), lambda i: (i, 0)),
            pl.BlockSpec((C, A), lambda i: (0, 0)),
            pl.BlockSpec((1, A), lambda i: (0, 0)),
            pl.BlockSpec((1, A), lambda i: (0, 0)),
        ],
        out_specs=[
            pl.BlockSpec((_BLK, A), lambda i: (i, 0)),
            pl.BlockSpec((1, A), lambda i: (0, 0)),
        ],
        out_shape=[
            jax.ShapeDtypeStruct((N, A), jnp.float32),
            jax.ShapeDtypeStruct((1, A), jnp.float32),
        ],
    )(h, W_out, b_out.reshape(1, A), log_std.reshape(1, A))
    return mu, sg.reshape(A)


def kernel(state, edge_index, edge_attr, W_in, b_in, W_g1, W_g2, W_out, b_out, log_std):
    src = edge_index[0]
    dst = edge_index[1]
    w = edge_attr

    pad = jnp.asarray(_np.arange(EPAD - E, dtype=_np.int32) % N)
    srcp = jnp.concatenate([src, pad]).reshape(NW, RPW, 128)
    dstp = jnp.concatenate([dst, pad]).reshape(NW, RPW, 128)
    wp = jnp.concatenate(
        [w, jnp.zeros((EPAD - E,), jnp.float32)]).reshape(NW, RPW, 128)
    zeros1 = jnp.zeros((NPAD,), jnp.float32)
    zeros2 = jnp.zeros((NPAD, C), jnp.float32)

    deg_p = _sc_deg(dstp, wp, zeros1)
    h = _read_in(state, W_in, b_in)
    dinv2d, rdinv2d = _deg_finish(deg_p)
    dinv_col = dinv2d.reshape(NPAD)[:N].reshape(N, 1)
    rdinv_col = rdinv2d.reshape(NPAD)[:N].reshape(N, 1)

    srcq = srcp.reshape(NW, -1, 64)
    dstq = dstp.reshape(NW, -1, 64)
    wq = wp.reshape(NW, -1, 64)

    g = _scale_rows(h, dinv_col)
    for Wt in (W_g1, W_g2):
        p1 = _sc_prop(g, srcq, dstq, wq, zeros2)
        g1 = _combine(p1, dinv_col)
        p2 = _sc_prop(g1, srcq, dstq, wq, zeros2)
        z = _zacc_first(h, g1, rdinv_col, Wt[0], Wt[1])
        g2 = _combine(p2, dinv_col)
        p3 = _sc_prop(g2, srcq, dstq, wq, zeros2)
        z = _zacc(z, g2, rdinv_col, Wt[2])
        g3 = _combine(p3, dinv_col)
        p4 = _sc_prop(g3, srcq, dstq, wq, zeros2)
        z = _zacc(z, g3, rdinv_col, Wt[3])
        h, g = _tap_last(p4, dinv_col, Wt[TAPS], z)

    return _read_out(h, W_out, b_out, log_std)


# speedup vs baseline: 1.0946x; 1.0072x over previous
import functools

import numpy as _np

import jax
import jax.numpy as jnp
from jax import lax
from jax.experimental import pallas as pl
from jax.experimental.pallas import tpu as pltpu
from jax.experimental.pallas import tpu_sc as plsc

N = 10000
E = 320000
D = 128
C = 128
A = 32
TAPS = 4

NC = 2
NS = 16
NW = NC * NS
NPAD = 10240
NSLC = NPAD // NS
RPW = 80
EPAD = NW * RPW * 128

_BLK = 2000
_GRID = N // _BLK


def _leaky(x):
    return jnp.where(x >= 0, x, 0.01 * x)


def _bcast_lane(v16, lane):
    idx = jnp.full((16, 1), lane, dtype=jnp.int32)
    return lax.gather(
        v16, idx,
        dimension_numbers=lax.GatherDimensionNumbers(
            offset_dims=(), collapsed_slice_dims=(0,), start_index_map=(0,)),
        slice_sizes=(1,),
        mode=lax.GatherScatterMode.PROMISE_IN_BOUNDS)


_MESH = plsc.VectorSubcoreMesh(core_axis_name="c", subcore_axis_name="s")


def _sc_deg(dstp, wp, zeros1):
    CR = 8

    @functools.partial(
        pl.kernel, mesh=_MESH,
        out_type=jax.ShapeDtypeStruct((NC, NPAD), jnp.float32),
        scratch_types=[
            pltpu.VMEM((CR, 128), jnp.int32),
            pltpu.VMEM((CR, 128), jnp.float32),
            pltpu.VMEM_SHARED((NPAD,), jnp.float32),
            pltpu.SemaphoreType.DMA,
            pltpu.SemaphoreType.DMA,
        ])
    def k(dst_hbm, w_hbm, z_hbm, out_hbm, didx, wv, acc, sem_a, sem_b):
        c = lax.axis_index("c")
        s = lax.axis_index("s")
        wid = s * NC + c
        pltpu.sync_copy(z_hbm.at[pl.ds(s * NSLC, NSLC)],
                        acc.at[pl.ds(s * NSLC, NSLC)])
        plsc.subcore_barrier()

        def chunk(ci, _):
            base = ci * CR
            st = [pltpu.async_copy(dst_hbm.at[wid, pl.ds(base, CR), :],
                                   didx, sem_a),
                  pltpu.async_copy(w_hbm.at[wid, pl.ds(base, CR), :],
                                   wv, sem_b)]
            for h_ in st:
                h_.wait()
            sc_ = [pltpu.async_copy(wv.at[j], acc.at[didx.at[j]], sem_a,
                                    add=True)
                   for j in range(CR)]
            for h_ in sc_:
                h_.wait()
            return 0

        lax.fori_loop(0, RPW // CR, chunk, 0)
        plsc.subcore_barrier()
        pltpu.sync_copy(acc.at[pl.ds(s * NSLC, NSLC)],
                        out_hbm.at[c, pl.ds(s * NSLC, NSLC)])

    return k(dstp, wp, zeros1)


def _sc_prop(h, srcp, dstp, normp, zeros2):
    NB = 4
    EW = 64
    SUPER = 40
    NSUB = EPAD // NW // EW
    NQ = SUPER // NB

    @functools.partial(
        pl.kernel, mesh=_MESH,
        out_type=jax.ShapeDtypeStruct((NC, NPAD, C), jnp.float32),
        scratch_types=[
            pltpu.VMEM((SUPER, EW), jnp.int32),
            pltpu.VMEM((SUPER, EW), jnp.int32),
            pltpu.VMEM((SUPER, EW), jnp.float32),
            [pltpu.VMEM((EW, C), jnp.float32) for _ in range(NB)],
            pltpu.VMEM_SHARED((NPAD, C), jnp.float32),
            [pltpu.SemaphoreType.DMA for _ in range(NB)],
            pltpu.SemaphoreType.DMA,
        ])
    def k(h_hbm, src_hbm, dst_hbm, nrm_hbm, z_hbm, out_hbm,
          sidx, didx, nrm, rows, acc, sem_g, sem_s):
        c = lax.axis_index("c")
        s = lax.axis_index("s")
        wid = s * NC + c

        def zrow(e, _):
            for q in range(8):
                rows[0][e, pl.ds(q * 16, 16)] = jnp.zeros((16,), jnp.float32)
            return 0

        lax.fori_loop(0, EW, zrow, 0)

        def zcp(j, _):
            pltpu.sync_copy(rows[0],
                            acc.at[pl.ds(s * NSLC + j * EW, EW), :])
            return 0

        lax.fori_loop(0, NSLC // EW, zcp, 0)
        plsc.subcore_barrier()

        def _drain_scatter():
            pltpu.make_async_copy(z_hbm.at[pl.ds(0, EW), :], rows[0],
                                  sem_s).wait()

        def _scale(buf, row):
            def grp(g, _):
                nv16 = nrm[row, pl.ds(g * 16, 16)]
                for l in range(16):
                    nb = _bcast_lane(nv16, l)
                    e = g * 16 + l
                    for q in range(8):
                        sl = pl.ds(q * 16, 16)
                        buf[e, sl] = buf[e, sl] * nb
                return 0
            lax.fori_loop(0, EW // 16, grp, 0)

        def superchunk(sc, _):
            rbase = sc * SUPER
            st = [pltpu.async_copy(src_hbm.at[wid, pl.ds(rbase, SUPER), :],
                                   sidx, sem_g[0]),
                  pltpu.async_copy(dst_hbm.at[wid, pl.ds(rbase, SUPER), :],
                                   didx, sem_g[1]),
                  pltpu.async_copy(nrm_hbm.at[wid, pl.ds(rbase, SUPER), :],
                                   nrm, sem_g[2])]
            for h_ in st:
                h_.wait()
            A = 2
            for u in range(A):
                pltpu.async_copy(h_hbm.at[sidx.at[u]], rows[u], sem_g[u])

            def quad(i, _):
                for u in range(NB):
                    t = i * NB + u
                    pltpu.make_async_copy(h_hbm.at[sidx.at[t]], rows[u],
                                          sem_g[u]).wait()

                    @pl.when(t >= NB - A)
                    def _():
                        _drain_scatter()

                    un = (u + A) % NB

                    @pl.when(t + A <= SUPER - 1)
                    def _():
                        pltpu.async_copy(h_hbm.at[sidx.at[t + A]],
                                         rows[un], sem_g[un])

                    _scale(rows[u], t)
                    pltpu.async_copy(rows[u], acc.at[didx.at[t]], sem_s,
                                     add=True)
                return 0

            lax.fori_loop(0, NQ, quad, 0)
            for _ in range(NB - A):
                _drain_scatter()
            return 0

        lax.fori_loop(0, NSUB // SUPER, superchunk, 0)
        plsc.subcore_barrier()
        pltpu.sync_copy(acc.at[pl.ds(s * NSLC, NSLC), :],
                        out_hbm.at[c, pl.ds(s * NSLC, NSLC), :])

    return k(h, srcp, dstp, normp, zeros2)


def _k_deg_fin(degp_ref, out_ref, rout_ref):
    x = degp_ref[...]
    t = jnp.maximum(x[:NPAD // 128, :] + x[NPAD // 128:, :], 1e-12)
    dv = lax.rsqrt(t)
    out_ref[...] = dv
    rout_ref[...] = jnp.sqrt(t)


def _deg_finish(deg_p):
    nr = NPAD // 128
    return pl.pallas_call(
        _k_deg_fin,
        in_specs=[pl.BlockSpec((2 * nr, 128), lambda: (0, 0))],
        out_specs=[
            pl.BlockSpec((nr, 128), lambda: (0, 0)),
            pl.BlockSpec((nr, 128), lambda: (0, 0)),
        ],
        out_shape=[
            jax.ShapeDtypeStruct((nr, 128), jnp.float32),
            jax.ShapeDtypeStruct((nr, 128), jnp.float32),
        ],
    )(deg_p.reshape(2 * nr, 128))


def _k_read_in(state_ref, w_ref, b_ref, h_ref):
    h_ref[...] = _leaky(
        jnp.dot(state_ref[...], w_ref[...], preferred_element_type=jnp.float32)
        + b_ref[...])


def _read_in(state, W_in, b_in):
    return pl.pallas_call(
        _k_read_in,
        grid=(_GRID,),
        in_specs=[
            pl.BlockSpec((_BLK, D), lambda i: (i, 0)),
            pl.BlockSpec((D, C), lambda i: (0, 0)),
            pl.BlockSpec((1, C), lambda i: (0, 0)),
        ],
        out_specs=pl.BlockSpec((_BLK, C), lambda i: (i, 0)),
        out_shape=jax.ShapeDtypeStruct((N, C), jnp.float32),
    )(state, W_in, b_in.reshape(1, C))


def _k_scale(x_ref, dv_ref, out_ref):
    out_ref[...] = dv_ref[...] * x_ref[...]


def _scale_rows(x, dinv_col):
    return pl.pallas_call(
        _k_scale,
        grid=(_GRID,),
        in_specs=[
            pl.BlockSpec((_BLK, C), lambda i: (i, 0)),
            pl.BlockSpec((_BLK, 1), lambda i: (i, 0)),
        ],
        out_specs=pl.BlockSpec((_BLK, C), lambda i: (i, 0)),
        out_shape=jax.ShapeDtypeStruct((N, C), jnp.float32),
    )(x, dinv_col)


def _p_specs():
    return [
        pl.BlockSpec((1, _BLK, C), lambda i: (0, i, 0)),
        pl.BlockSpec((1, _BLK, C), lambda i: (1, i, 0)),
    ]


def _k_combine(p0_ref, p1_ref, dv_ref, gk_ref):
    dv = dv_ref[...]
    gk_ref[...] = dv * dv * (p0_ref[0] + p1_ref[0])


def _combine(p, dinv_col):
    return pl.pallas_call(
        _k_combine,
        grid=(_GRID,),
        in_specs=[*_p_specs(), pl.BlockSpec((_BLK, 1), lambda i: (i, 0))],
        out_specs=pl.BlockSpec((_BLK, C), lambda i: (i, 0)),
        out_shape=jax.ShapeDtypeStruct((N, C), jnp.float32),
    )(p, p, dinv_col)


def _k_zacc_first(h_ref, g1_ref, rv_ref, w0_ref, w1_ref, z_ref):
    h1 = rv_ref[...] * g1_ref[...]
    z_ref[...] = (
        jnp.dot(h_ref[...], w0_ref[...], preferred_element_type=jnp.float32)
        + jnp.dot(h1, w1_ref[...], preferred_element_type=jnp.float32)
    )


def _zacc_first(h, g1, rdinv_col, W0, W1):
    return pl.pallas_call(
        _k_zacc_first,
        grid=(_GRID,),
        in_specs=[
            pl.BlockSpec((_BLK, C), lambda i: (i, 0)),
            pl.BlockSpec((_BLK, C), lambda i: (i, 0)),
            pl.BlockSpec((_BLK, 1), lambda i: (i, 0)),
            pl.BlockSpec((C, C), lambda i: (0, 0)),
            pl.BlockSpec((C, C), lambda i: (0, 0)),
        ],
        out_specs=pl.BlockSpec((_BLK, C), lambda i: (i, 0)),
        out_shape=jax.ShapeDtypeStruct((N, C), jnp.float32),
    )(h, g1, rdinv_col, W0, W1)


def _k_zacc(z_ref, gk_ref, rv_ref, w_ref, z2_ref):
    hk = rv_ref[...] * gk_ref[...]
    z2_ref[...] = z_ref[...] + jnp.dot(
        hk, w_ref[...], preferred_element_type=jnp.float32)


def _zacc(z, gk, rdinv_col, W):
    return pl.pallas_call(
        _k_zacc,
        grid=(_GRID,),
        in_specs=[
            pl.BlockSpec((_BLK, C), lambda i: (i, 0)),
            pl.BlockSpec((_BLK, C), lambda i: (i, 0)),
            pl.BlockSpec((_BLK, 1), lambda i: (i, 0)),
            pl.BlockSpec((C, C), lambda i: (0, 0)),
        ],
        out_specs=pl.BlockSpec((_BLK, C), lambda i: (i, 0)),
        out_shape=jax.ShapeDtypeStruct((N, C), jnp.float32),
    )(z, gk, rdinv_col, W)


def _k_tap_last(p0_ref, p1_ref, dv_ref, w_ref, z_ref, h_ref, g_ref):
    dv = dv_ref[...]
    hk = dv * (p0_ref[0] + p1_ref[0])
    h = _leaky(z_ref[...] + jnp.dot(
        hk, w_ref[...], preferred_element_type=jnp.float32))
    h_ref[...] = h
    g_ref[...] = dv * h


def _tap_last(p, dinv_col, W, z):
    return pl.pallas_call(
        _k_tap_last,
        grid=(_GRID,),
        in_specs=[
            *_p_specs(),
            pl.BlockSpec((_BLK, 1), lambda i: (i, 0)),
            pl.BlockSpec((C, C), lambda i: (0, 0)),
            pl.BlockSpec((_BLK, C), lambda i: (i, 0)),
        ],
        out_specs=[
            pl.BlockSpec((_BLK, C), lambda i: (i, 0)),
            pl.BlockSpec((_BLK, C), lambda i: (i, 0)),
        ],
        out_shape=[
            jax.ShapeDtypeStruct((N, C), jnp.float32),
            jax.ShapeDtypeStruct((N, C), jnp.float32),
        ],
    )(p, p, dinv_col, W, z)


def _k_read_out(h_ref, w_ref, b_ref, ls_ref, mu_ref, sg_ref):
    mu_ref[...] = jnp.dot(
        h_ref[...], w_ref[...], preferred_element_type=jnp.float32
    ) + b_ref[...]
    sg_ref[...] = jnp.exp(ls_ref[...])


def _read_out(h, W_out, b_out, log_std):
    mu, sg = pl.pallas_call(
        _k_read_out,
        grid=(_GRID,),
        in_specs=[
            pl.BlockSpec((_BLK, C), lambda i: (i, 0)),
            pl.BlockSpec((C, A), lambda i: (0, 0)),
            pl.BlockSpec((1, A), lambda i: (0, 0)),
            pl.BlockSpec((1, A), lambda i: (0, 0)),
        ],
        out_specs=[
            pl.BlockSpec((_BLK, A), lambda i: (i, 0)),
            pl.BlockSpec((1, A), lambda i: (0, 0)),
        ],
        out_shape=[
            jax.ShapeDtypeStruct((N, A), jnp.float32),
            jax.ShapeDtypeStruct((1, A), jnp.float32),
        ],
    )(h, W_out, b_out.reshape(1, A), log_std.reshape(1, A))
    return mu, sg.reshape(A)


def kernel(state, edge_index, edge_attr, W_in, b_in, W_g1, W_g2, W_out, b_out, log_std):
    src = edge_index[0]
    dst = edge_index[1]
    w = edge_attr

    pad = jnp.asarray(_np.arange(EPAD - E, dtype=_np.int32) % N)
    srcp = jnp.concatenate([src, pad]).reshape(NW, RPW, 128)
    dstp = jnp.concatenate([dst, pad]).reshape(NW, RPW, 128)
    wp = jnp.concatenate(
        [w, jnp.zeros((EPAD - E,), jnp.float32)]).reshape(NW, RPW, 128)
    zeros1 = jnp.zeros((NPAD,), jnp.float32)
    zeros2 = jnp.zeros((NPAD, C), jnp.float32)

    deg_p = _sc_deg(dstp, wp, zeros1)
    h = _read_in(state, W_in, b_in)
    dinv2d, rdinv2d = _deg_finish(deg_p)
    dinv_col = dinv2d.reshape(NPAD)[:N].reshape(N, 1)
    rdinv_col = rdinv2d.reshape(NPAD)[:N].reshape(N, 1)

    srcq = srcp.reshape(NW, -1, 64)
    dstq = dstp.reshape(NW, -1, 64)
    wq = wp.reshape(NW, -1, 64)

    g = _scale_rows(h, dinv_col)
    for Wt in (W_g1, W_g2):
        p1 = _sc_prop(g, srcq, dstq, wq, zeros2)
        g1 = _combine(p1, dinv_col)
        p2 = _sc_prop(g1, srcq, dstq, wq, zeros2)
        z = _zacc_first(h, g1, rdinv_col, Wt[0], Wt[1])
        g2 = _combine(p2, dinv_col)
        p3 = _sc_prop(g2, srcq, dstq, wq, zeros2)
        z = _zacc(z, g2, rdinv_col, Wt[2])
        g3 = _combine(p3, dinv_col)
        p4 = _sc_prop(g3, srcq, dstq, wq, zeros2)
        z = _zacc(z, g3, rdinv_col, Wt[3])
        h, g = _tap_last(p4, dinv_col, Wt[TAPS], z)

    return _read_out(h, W_out, b_out, log_std)
